# Initial kernel scaffold; baseline (speedup 1.0000x reference)
import jax
import jax.numpy as jnp
from jax.experimental import pallas as pl


def kernel(tokens, table):
    raise NotImplementedError("write your pallas kernel here")


# speedup vs baseline: 1.4165x; 1.4165x over previous
import functools
import math

import jax
import jax.numpy as jnp
from jax import lax
from jax.experimental import pallas as pl
from jax.experimental.pallas import tpu as pltpu
from jax.experimental.pallas import tpu_sc as plsc

_EMB = 32
_SCALE = math.sqrt(_EMB)

_NC = 2
_NS = 16
_NW = _NC * _NS

_B = 4096 * 200
_BPW = _B // _NW
_CHUNK = 1600
_NCHUNK = _BPW // _CHUNK
_ROW_UNROLL = 4


def _emb_kernel(tok_hbm, table_hbm, out_hbm, idx_v, rows_v, sem):
    wid = lax.axis_index("s") * _NC + lax.axis_index("c")
    base = wid * _BPW

    def chunk_body(ci, carry):
        off = base + ci * _CHUNK
        pltpu.sync_copy(tok_hbm.at[pl.ds(off, _CHUNK)], idx_v)
        pltpu.async_copy(table_hbm.at[idx_v], rows_v, sem).wait()

        def row_body(i, c2):
            r = i * _ROW_UNROLL
            for u in range(_ROW_UNROLL):
                rows_v[r + u, pl.ds(0, 16)] = rows_v[r + u, pl.ds(0, 16)] * _SCALE
                rows_v[r + u, pl.ds(16, 16)] = rows_v[r + u, pl.ds(16, 16)] * _SCALE
            return c2

        lax.fori_loop(0, _CHUNK // _ROW_UNROLL, row_body, 0)
        pltpu.sync_copy(rows_v, out_hbm.at[pl.ds(off, _CHUNK)])
        return carry

    lax.fori_loop(0, _NCHUNK, chunk_body, 0)


@jax.jit
def _lookup(flat_tokens, table):
    mesh = plsc.VectorSubcoreMesh(core_axis_name="c", subcore_axis_name="s")
    run = functools.partial(
        pl.kernel,
        mesh=mesh,
        out_type=jax.ShapeDtypeStruct((_B, _EMB), jnp.float32),
        scratch_types=[
            pltpu.VMEM((_CHUNK,), jnp.int32),
            pltpu.VMEM((_CHUNK, _EMB), jnp.float32),
            pltpu.SemaphoreType.DMA,
        ],
        compiler_params=pltpu.CompilerParams(use_tc_tiling_on_sc=False),
    )(_emb_kernel)
    return run(flat_tokens, table)


def kernel(tokens, table):
    flat = tokens.reshape(-1).astype(jnp.int32)
    out = _lookup(flat, table)
    return out.reshape(tokens.shape + (_EMB,))


# speedup vs baseline: 1.4596x; 1.0304x over previous
import functools
import math

import jax
import jax.numpy as jnp
from jax import lax
from jax.experimental import pallas as pl
from jax.experimental.pallas import tpu as pltpu
from jax.experimental.pallas import tpu_sc as plsc

_EMB = 32
_SCALE = math.sqrt(_EMB)

_NC = 2
_NS = 16
_NW = _NC * _NS

_B = 4096 * 200
_BPW = _B // _NW
_CHUNK = 800
_NCHUNK = _BPW // _CHUNK
_NBUF = 4
_ROW_UNROLL = 4


def _emb_kernel(tok_hbm, table_hbm, out_hbm, idx_v, rows_v, gsem, osem):
    wid = lax.axis_index("s") * _NC + lax.axis_index("c")
    base = wid * _BPW

    def start_gather(ci, b):
        off = base + ci * _CHUNK
        pltpu.sync_copy(tok_hbm.at[pl.ds(off, _CHUNK)], idx_v.at[b])
        pltpu.async_copy(table_hbm.at[idx_v.at[b]], rows_v.at[b], gsem.at[b])

    def wait_gather(b):
        pltpu.make_async_copy(
            table_hbm.at[idx_v.at[b]], rows_v.at[b], gsem.at[b]).wait()

    def start_write(ci, b):
        off = base + ci * _CHUNK
        pltpu.make_async_copy(
            rows_v.at[b], out_hbm.at[pl.ds(off, _CHUNK)], osem.at[b]).start()

    def wait_write(b):
        pltpu.make_async_copy(
            rows_v.at[b], out_hbm.at[pl.ds(base, _CHUNK)], osem.at[b]).wait()

    def scale(b):
        def row_body(i, c2):
            r = i * _ROW_UNROLL
            for u in range(_ROW_UNROLL):
                rows_v[b, r + u, pl.ds(0, 16)] = (
                    rows_v[b, r + u, pl.ds(0, 16)] * _SCALE)
                rows_v[b, r + u, pl.ds(16, 16)] = (
                    rows_v[b, r + u, pl.ds(16, 16)] * _SCALE)
            return c2
        lax.fori_loop(0, _CHUNK // _ROW_UNROLL, row_body, 0)

    for b in range(_NBUF - 1):
        start_gather(b, b)

    def group_body(g, carry):
        for b in range(_NBUF):
            ci = g * _NBUF + b
            bprev = (b + _NBUF - 1) % _NBUF

            @pl.when(ci > 0)
            def _():
                wait_write(bprev)

            @pl.when(ci + _NBUF - 1 < _NCHUNK)
            def _():
                start_gather(ci + _NBUF - 1, bprev)

            wait_gather(b)
            scale(b)
            start_write(ci, b)
        return carry

    lax.fori_loop(0, _NCHUNK // _NBUF, group_body, 0)
    wait_write((_NCHUNK - 1) % _NBUF)


@jax.jit
def _lookup(flat_tokens, table):
    mesh = plsc.VectorSubcoreMesh(core_axis_name="c", subcore_axis_name="s")
    run = functools.partial(
        pl.kernel,
        mesh=mesh,
        out_type=jax.ShapeDtypeStruct((_B, _EMB), jnp.float32),
        scratch_types=[
            pltpu.VMEM((_NBUF, _CHUNK), jnp.int32),
            pltpu.VMEM((_NBUF, _CHUNK, _EMB), jnp.float32),
            pltpu.SemaphoreType.DMA((_NBUF,)),
            pltpu.SemaphoreType.DMA((_NBUF,)),
        ],
        compiler_params=pltpu.CompilerParams(use_tc_tiling_on_sc=False),
    )(_emb_kernel)
    return run(flat_tokens, table)


def kernel(tokens, table):
    flat = tokens.reshape(-1).astype(jnp.int32)
    out = _lookup(flat, table)
    return out.reshape(tokens.shape + (_EMB,))
